# baseline (speedup 1.0000x reference)
import functools

import jax
import jax.numpy as jnp
from jax import lax
from jax.experimental import pallas as pl
from jax.experimental.pallas import tpu as pltpu
from jax.experimental.pallas import tpu_sc as plsc

_B = 16384
_NC, _NS, _L = 2, 16, 16
_NW = _NC * _NS
_BPW = _B // _NW
_BC = 128
_JPW = _BPW // _BC
_NH = 9
_NU = _NH * _JPW

_mesh = plsc.VectorSubcoreMesh(core_axis_name="c", subcore_axis_name="s")


@functools.partial(
    pl.kernel,
    mesh=_mesh,
    out_type=jax.ShapeDtypeStruct((9, 9, 9, _B), jnp.float32),
    scratch_types=[
        pltpu.VMEM((9, 9, _BC), jnp.float32),
        pltpu.VMEM((9, 9, _BC), jnp.float32),
        pltpu.VMEM((9, 9, _BC), jnp.float32),
        pltpu.VMEM((9, 9, _BC), jnp.float32),
        pltpu.VMEM((9, 9, _BC), jnp.float32),
        pltpu.VMEM((9, 9, _BC), jnp.float32),
        pltpu.SemaphoreType.DMA,
        pltpu.SemaphoreType.DMA,
        pltpu.SemaphoreType.DMA,
        pltpu.SemaphoreType.DMA,
        pltpu.SemaphoreType.DMA,
        pltpu.SemaphoreType.DMA,
    ],
    compiler_params=pltpu.CompilerParams(use_tc_tiling_on_sc=True),
)
def _sc_select(s_hbm, m_hbm, o_hbm, s0, s1, m0, m1, o0, o1,
               ss0, ss1, sm0, sm1, so0, so1):
    wid = lax.axis_index("s") * _NC + lax.axis_index("c")
    wbase = wid * _BPW
    sb, mb, ob = (s0, s1), (m0, m1), (o0, o1)
    ssem, msem, osem = (ss0, ss1), (sm0, sm1), (so0, so1)

    def slab(hbm, u):
        h = u // _JPW
        boff = wbase + (u % _JPW) * _BC
        return hbm.at[:, h, :, pl.ds(boff, _BC)]

    def compute(sv, mv, ov):
        for w in range(9):
            def lbody(l2, c2, w=w):
                for t in range(2):
                    sl = pl.ds((l2 * 2 + t) * _L, _L)
                    v = [sv[c, w, sl] for c in range(9)]
                    mx = v[0]
                    idxf = jnp.zeros((_L,), jnp.float32)
                    for c in range(1, 9):
                        gt = v[c] > mx
                        mx = jnp.maximum(mx, v[c])
                        idxf = jnp.where(gt, jnp.float32(c), idxf)
                    for c in range(9):
                        out = jnp.where(idxf == jnp.float32(c),
                                        v[c] * mv[c, w, sl], 0.0)
                        ov[c, w, sl] = out
                return c2

            lax.fori_loop(0, _BC // _L // 2, lbody, 0)

    for k in (0, 1):
        pltpu.make_async_copy(slab(s_hbm, k), sb[k], ssem[k]).start()
        pltpu.make_async_copy(slab(m_hbm, k), mb[k], msem[k]).start()

    def gloop(gp, carry):
        for k in (0, 1):
            u = gp * 2 + k
            pltpu.make_async_copy(slab(s_hbm, u), sb[k], ssem[k]).wait()
            pltpu.make_async_copy(slab(m_hbm, u), mb[k], msem[k]).wait()

            @pl.when(u >= 2)
            def _wait_out():
                pltpu.make_async_copy(ob[k], slab(o_hbm, u), osem[k]).wait()

            compute(sb[k], mb[k], ob[k])
            pltpu.make_async_copy(ob[k], slab(o_hbm, u), osem[k]).start()

            @pl.when(u + 2 < _NU)
            def _next_in():
                pltpu.make_async_copy(slab(s_hbm, u + 2), sb[k], ssem[k]).start()
                pltpu.make_async_copy(slab(m_hbm, u + 2), mb[k], msem[k]).start()
        return carry

    lax.fori_loop(0, _NU // 2, gloop, 0)

    for k in (0, 1):
        u = _NU - 2 + k
        pltpu.make_async_copy(ob[k], slab(o_hbm, u), osem[k]).wait()


def kernel(sudoku, current_cell_mask):
    st = jnp.transpose(sudoku, (1, 2, 3, 0))
    mt = jnp.transpose(current_cell_mask, (1, 2, 3, 0))
    ot = _sc_select(st, mt)
    return jnp.transpose(ot, (3, 0, 1, 2))


# speedup vs baseline: 1.3407x; 1.3407x over previous
import functools

import jax
import jax.numpy as jnp
from jax import lax
from jax.experimental import pallas as pl
from jax.experimental.pallas import tpu as pltpu
from jax.experimental.pallas import tpu_sc as plsc

_B = 16384
_NC, _NS, _L = 2, 16, 16
_NW = _NC * _NS
_BPW = _B // _NW
_BC = 128
_JPW = _BPW // _BC
_NH = 9
_NU = _NH * _JPW

_mesh = plsc.VectorSubcoreMesh(core_axis_name="c", subcore_axis_name="s")


@functools.partial(
    pl.kernel,
    mesh=_mesh,
    out_type=jax.ShapeDtypeStruct((9, 9, 9, _B), jnp.float32),
    scratch_types=[
        pltpu.VMEM((9, 9, _BC), jnp.float32),
        pltpu.VMEM((9, 9, _BC), jnp.float32),
        pltpu.VMEM((9, 9, _BC), jnp.float32),
        pltpu.VMEM((9, 9, _BC), jnp.float32),
        pltpu.VMEM((9, 9, _BC), jnp.float32),
        pltpu.VMEM((9, 9, _BC), jnp.float32),
        pltpu.SemaphoreType.DMA,
        pltpu.SemaphoreType.DMA,
        pltpu.SemaphoreType.DMA,
        pltpu.SemaphoreType.DMA,
        pltpu.SemaphoreType.DMA,
        pltpu.SemaphoreType.DMA,
    ],
    compiler_params=pltpu.CompilerParams(use_tc_tiling_on_sc=True),
)
def _sc_select(s_hbm, m_hbm, o_hbm, s0, s1, m0, m1, o0, o1,
               ss0, ss1, sm0, sm1, so0, so1):
    wid = lax.axis_index("s") * _NC + lax.axis_index("c")
    wbase = wid * _BPW
    sb, mb, ob = (s0, s1), (m0, m1), (o0, o1)
    ssem, msem, osem = (ss0, ss1), (sm0, sm1), (so0, so1)

    def slab(hbm, u):
        h = u // _JPW
        boff = wbase + (u % _JPW) * _BC
        return hbm.at[:, h, :, pl.ds(boff, _BC)]

    def compute(sv, mv, ov):
        for w in range(9):
            @plsc.parallel_loop(0, _BC // _L)
            def lbody(l, w=w):
                sl = pl.ds(l * _L, _L)
                v = [sv[c, w, sl] for c in range(9)]
                mx = v[0]
                idxf = jnp.zeros((_L,), jnp.float32)
                for c in range(1, 9):
                    gt = v[c] > mx
                    mx = jnp.maximum(mx, v[c])
                    idxf = jnp.where(gt, jnp.float32(c), idxf)
                for c in range(9):
                    out = jnp.where(idxf == jnp.float32(c),
                                    v[c] * mv[c, w, sl], 0.0)
                    ov[c, w, sl] = out

    for k in (0, 1):
        pltpu.make_async_copy(slab(s_hbm, k), sb[k], ssem[k]).start()
        pltpu.make_async_copy(slab(m_hbm, k), mb[k], msem[k]).start()

    def gloop(gp, carry):
        for k in (0, 1):
            u = gp * 2 + k
            pltpu.make_async_copy(slab(s_hbm, u), sb[k], ssem[k]).wait()
            pltpu.make_async_copy(slab(m_hbm, u), mb[k], msem[k]).wait()

            @pl.when(u >= 2)
            def _wait_out():
                pltpu.make_async_copy(ob[k], slab(o_hbm, u), osem[k]).wait()

            compute(sb[k], mb[k], ob[k])
            pltpu.make_async_copy(ob[k], slab(o_hbm, u), osem[k]).start()

            @pl.when(u + 2 < _NU)
            def _next_in():
                pltpu.make_async_copy(slab(s_hbm, u + 2), sb[k], ssem[k]).start()
                pltpu.make_async_copy(slab(m_hbm, u + 2), mb[k], msem[k]).start()
        return carry

    lax.fori_loop(0, _NU // 2, gloop, 0)

    for k in (0, 1):
        u = _NU - 2 + k
        pltpu.make_async_copy(ob[k], slab(o_hbm, u), osem[k]).wait()


def kernel(sudoku, current_cell_mask):
    st = jnp.transpose(sudoku, (1, 2, 3, 0))
    mt = jnp.transpose(current_cell_mask, (1, 2, 3, 0))
    ot = _sc_select(st, mt)
    return jnp.transpose(ot, (3, 0, 1, 2))


# speedup vs baseline: 1.3915x; 1.0379x over previous
import functools

import jax
import jax.numpy as jnp
from jax import lax
from jax.experimental import pallas as pl
from jax.experimental.pallas import tpu as pltpu
from jax.experimental.pallas import tpu_sc as plsc

_B = 16384
_NC, _NS, _L = 2, 16, 16
_NW = _NC * _NS
_BPW = _B // _NW
_BC = 128
_JPW = _BPW // _BC
_NH = 9
_NU = _NH * _JPW

_mesh = plsc.VectorSubcoreMesh(core_axis_name="c", subcore_axis_name="s")


@functools.partial(
    pl.kernel,
    mesh=_mesh,
    out_type=jax.ShapeDtypeStruct((9, 9, 9, _B), jnp.float32),
    scratch_types=[
        pltpu.VMEM((9, 9, _BC), jnp.float32),
        pltpu.VMEM((9, 9, _BC), jnp.float32),
        pltpu.VMEM((9, 9, _BC), jnp.float32),
        pltpu.VMEM((9, 9, _BC), jnp.float32),
        pltpu.VMEM((9, 9, _BC), jnp.float32),
        pltpu.VMEM((9, 9, _BC), jnp.float32),
        pltpu.SemaphoreType.DMA,
        pltpu.SemaphoreType.DMA,
        pltpu.SemaphoreType.DMA,
        pltpu.SemaphoreType.DMA,
        pltpu.SemaphoreType.DMA,
        pltpu.SemaphoreType.DMA,
    ],
    compiler_params=pltpu.CompilerParams(use_tc_tiling_on_sc=True),
)
def _sc_select(s_hbm, m_hbm, o_hbm, s0, s1, m0, m1, o0, o1,
               ss0, ss1, sm0, sm1, so0, so1):
    wid = lax.axis_index("s") * _NC + lax.axis_index("c")
    wbase = wid * _BPW
    sb, mb, ob = (s0, s1), (m0, m1), (o0, o1)
    ssem, msem, osem = (ss0, ss1), (sm0, sm1), (so0, so1)

    def slab(hbm, u):
        u = lax.rem(u + wid, _NU)
        h = u // _JPW
        boff = wbase + (u % _JPW) * _BC
        return hbm.at[:, h, :, pl.ds(boff, _BC)]

    _NL = _BC // _L

    def compute(sv, mv, ov):
        @plsc.parallel_loop(0, 9 * _NL)
        def gbody(g):
            w = g // _NL
            sl = pl.ds((g % _NL) * _L, _L)
            v = [sv[c, w, sl] for c in range(9)]
            mx = v[0]
            idxf = jnp.zeros((_L,), jnp.float32)
            for c in range(1, 9):
                gt = v[c] > mx
                mx = jnp.maximum(mx, v[c])
                idxf = jnp.where(gt, jnp.float32(c), idxf)
            for c in range(9):
                out = jnp.where(idxf == jnp.float32(c),
                                v[c] * mv[c, w, sl], 0.0)
                ov[c, w, sl] = out

    for k in (0, 1):
        pltpu.make_async_copy(slab(s_hbm, k), sb[k], ssem[k]).start()
        pltpu.make_async_copy(slab(m_hbm, k), mb[k], msem[k]).start()

    def gloop(gp, carry):
        for k in (0, 1):
            u = gp * 2 + k
            pltpu.make_async_copy(slab(s_hbm, u), sb[k], ssem[k]).wait()
            pltpu.make_async_copy(slab(m_hbm, u), mb[k], msem[k]).wait()

            @pl.when(u >= 2)
            def _wait_out():
                pltpu.make_async_copy(ob[k], slab(o_hbm, u), osem[k]).wait()

            compute(sb[k], mb[k], ob[k])
            pltpu.make_async_copy(ob[k], slab(o_hbm, u), osem[k]).start()

            @pl.when(u + 2 < _NU)
            def _next_in():
                pltpu.make_async_copy(slab(s_hbm, u + 2), sb[k], ssem[k]).start()
                pltpu.make_async_copy(slab(m_hbm, u + 2), mb[k], msem[k]).start()
        return carry

    lax.fori_loop(0, _NU // 2, gloop, 0)

    for k in (0, 1):
        u = _NU - 2 + k
        pltpu.make_async_copy(ob[k], slab(o_hbm, u), osem[k]).wait()


def kernel(sudoku, current_cell_mask):
    st = jnp.transpose(sudoku, (1, 2, 3, 0))
    mt = jnp.transpose(current_cell_mask, (1, 2, 3, 0))
    ot = _sc_select(st, mt)
    return jnp.transpose(ot, (3, 0, 1, 2))
